# Initial kernel scaffold; baseline (speedup 1.0000x reference)
import jax
import jax.numpy as jnp
from jax.experimental import pallas as pl


def kernel(input_ids, word_emb, pos_emb, ln_gamma, ln_beta):
    raise NotImplementedError("write your pallas kernel here")


# speedup vs baseline: 1.4457x; 1.4457x over previous
import functools

import jax
import jax.numpy as jnp
from jax import lax
from jax.experimental import pallas as pl
from jax.experimental.pallas import tpu as pltpu
from jax.experimental.pallas import tpu_sc as plsc

EPS = 1e-12


def _make_sc_gather(V, D, B):
    info = plsc.get_sparse_core_info()
    NC, NS = info.num_cores, info.num_subcores
    NW = NC * NS
    assert B % NW == 0
    b_per_w = B // NW
    C = min(b_per_w, 64)
    assert b_per_w % C == 0
    n_chunks = b_per_w // C
    mesh = plsc.VectorSubcoreMesh(core_axis_name="c", subcore_axis_name="s")

    @functools.partial(
        pl.kernel,
        mesh=mesh,
        out_type=jax.ShapeDtypeStruct((B, D), jnp.float32),
        scratch_types=[
            pltpu.VMEM((C,), jnp.int32),
            pltpu.VMEM((C, D), jnp.float32),
            pltpu.SemaphoreType.DMA,
        ],
    )
    def sc_gather(table_hbm, idx_hbm, out_hbm, idx_v, rows_v, sem):
        wid = lax.axis_index("s") * NC + lax.axis_index("c")
        base = wid * b_per_w
        for c in range(n_chunks):
            lo = base + c * C
            pltpu.sync_copy(idx_hbm.at[pl.ds(lo, C)], idx_v)
            pltpu.async_copy(table_hbm.at[idx_v], rows_v, sem).wait()
            pltpu.sync_copy(rows_v, out_hbm.at[pl.ds(lo, C)])

    return sc_gather


def _tc_add_ln_body(g_ref, p_ref, gamma_ref, beta_ref, o_ref):
    x = g_ref[...] + p_ref[...]
    mean = jnp.mean(x, axis=-1, keepdims=True)
    xc = x - mean
    var = jnp.mean(xc * xc, axis=-1, keepdims=True)
    xhat = xc * lax.rsqrt(var + EPS)
    o_ref[...] = xhat * gamma_ref[...] + beta_ref[...]


def _tc_add_ln(gathered, pos_emb, gamma, beta):
    B, D = gathered.shape
    S = pos_emb.shape[0]
    R = 512
    n_blocks = B // R
    pos_blocks = S // R
    return pl.pallas_call(
        _tc_add_ln_body,
        grid=(n_blocks,),
        in_specs=[
            pl.BlockSpec((R, D), lambda i: (i, 0)),
            pl.BlockSpec((R, D), lambda i: (i % pos_blocks, 0)),
            pl.BlockSpec((1, D), lambda i: (0, 0)),
            pl.BlockSpec((1, D), lambda i: (0, 0)),
        ],
        out_specs=pl.BlockSpec((R, D), lambda i: (i, 0)),
        out_shape=jax.ShapeDtypeStruct((B, D), jnp.float32),
    )(gathered, pos_emb, gamma.reshape(1, D), beta.reshape(1, D))


def kernel(input_ids, word_emb, pos_emb, ln_gamma, ln_beta):
    Bt, S = input_ids.shape
    V, D = word_emb.shape
    ids = input_ids.reshape(-1).astype(jnp.int32)
    gathered = _make_sc_gather(V, D, Bt * S)(word_emb, ids)
    out = _tc_add_ln(gathered, pos_emb, ln_gamma, ln_beta)
    return out.reshape(Bt, S, D)


# speedup vs baseline: 1.5100x; 1.0445x over previous
import functools

import jax
import jax.numpy as jnp
from jax import lax
from jax.experimental import pallas as pl
from jax.experimental.pallas import tpu as pltpu
from jax.experimental.pallas import tpu_sc as plsc

EPS = 1e-12


def _make_sc_gather(V, D, B):
    info = plsc.get_sparse_core_info()
    NC, NS = info.num_cores, info.num_subcores
    NW = NC * NS
    assert B % NW == 0
    b_per_w = B // NW
    C = min(b_per_w, 64)
    assert b_per_w % C == 0
    n_chunks = b_per_w // C
    mesh = plsc.VectorSubcoreMesh(core_axis_name="c", subcore_axis_name="s")

    @functools.partial(
        pl.kernel,
        mesh=mesh,
        out_type=jax.ShapeDtypeStruct((B, D), jnp.float32),
        scratch_types=[
            pltpu.VMEM((C,), jnp.int32),
            pltpu.VMEM((C, D), jnp.float32),
            pltpu.SemaphoreType.DMA,
        ],
    )
    def sc_gather(table_hbm, idx_hbm, out_hbm, idx_v, rows_v, sem):
        wid = lax.axis_index("s") * NC + lax.axis_index("c")
        base = wid * b_per_w
        for c in range(n_chunks):
            lo = base + c * C
            pltpu.sync_copy(idx_hbm.at[pl.ds(lo, C)], idx_v)
            pltpu.async_copy(table_hbm.at[idx_v], rows_v, sem).wait()
            pltpu.sync_copy(rows_v, out_hbm.at[pl.ds(lo, C)])

    return sc_gather


def _tc_add_ln_body(g_ref, p_ref, gamma_ref, beta_ref, o_ref):
    x = g_ref[...] + p_ref[...][None, :, :]
    mean = jnp.mean(x, axis=-1, keepdims=True)
    xc = x - mean
    var = jnp.mean(xc * xc, axis=-1, keepdims=True)
    xhat = xc * lax.rsqrt(var + EPS)
    o_ref[...] = xhat * gamma_ref[...] + beta_ref[...]


def _tc_add_ln(gathered3, pos_emb, gamma, beta):
    Bt, S, D = gathered3.shape
    R = 512
    pos_blocks = S // R
    return pl.pallas_call(
        _tc_add_ln_body,
        grid=(pos_blocks, Bt),
        in_specs=[
            pl.BlockSpec((1, R, D), lambda j, b: (b, j, 0)),
            pl.BlockSpec((R, D), lambda j, b: (j, 0)),
            pl.BlockSpec((1, D), lambda j, b: (0, 0)),
            pl.BlockSpec((1, D), lambda j, b: (0, 0)),
        ],
        out_specs=pl.BlockSpec((1, R, D), lambda j, b: (b, j, 0)),
        out_shape=jax.ShapeDtypeStruct((Bt, S, D), jnp.float32),
    )(gathered3, pos_emb, gamma.reshape(1, D), beta.reshape(1, D))


def kernel(input_ids, word_emb, pos_emb, ln_gamma, ln_beta):
    Bt, S = input_ids.shape
    V, D = word_emb.shape
    ids = input_ids.reshape(-1).astype(jnp.int32)
    gathered = _make_sc_gather(V, D, Bt * S)(word_emb, ids)
    return _tc_add_ln(gathered.reshape(Bt, S, D), pos_emb, ln_gamma, ln_beta)
